# baseline (speedup 1.0000x reference)
import jax
import jax.numpy as jnp
from jax import lax
from jax.experimental import pallas as pl
from jax.experimental.pallas import tpu as pltpu
from jax.experimental.pallas import tpu_sc as plsc

B = 16384
L = 20
D = 64
DC = 128
TOK = 64

NC = 2
NS = 16
NW = NC * NS
PB = B // NW
CB = 32
NCH = PB // CB
RPC = CB * L
GID = 128
NG = RPC // GID


def _sc_body(idx_hbm, mask_hbm, llmidx_hbm, tool_tab, llm_tab,
             ids_out,
             idx_v, mask_v, lidx_v, rows_v, tm_v, lrows_v, gsems):
    c = lax.axis_index("c")
    s = lax.axis_index("s")
    wid = s * NC + c
    wbase = wid * PB

    pltpu.sync_copy(idx_hbm.at[pl.ds(wbase * L, PB * L)], idx_v)
    pltpu.sync_copy(mask_hbm.at[:, pl.ds(wbase, PB)], mask_v)
    pltpu.sync_copy(llmidx_hbm.at[pl.ds(wbase, PB)], lidx_v)

    def fire(buf, ci):
        o = ci * RPC
        for j in range(NG):
            pltpu.async_copy(
                tool_tab.at[idx_v.at[pl.ds(o + j * GID, GID)]],
                rows_v.at[buf, pl.ds(j * GID, GID)], gsems.at[buf])
        pltpu.async_copy(llm_tab.at[lidx_v.at[pl.ds(ci * CB, CB)]],
                         lrows_v.at[buf], gsems.at[buf])

    def drain(buf, ci):
        o = ci * RPC
        for j in range(NG):
            pltpu.make_async_copy(
                tool_tab.at[idx_v.at[pl.ds(o + j * GID, GID)]],
                rows_v.at[buf, pl.ds(j * GID, GID)], gsems.at[buf]).wait()
        pltpu.make_async_copy(llm_tab.at[lidx_v.at[pl.ds(ci * CB, CB)]],
                              lrows_v.at[buf], gsems.at[buf]).wait()

    def compute(buf, ci):
        base = wbase + ci * CB

        def gbody(g16, bc):
            o = ci * CB + g16 * 16
            ms = [mask_v[l, pl.ds(o, 16)] for l in range(L)]
            dsum = ms[0]
            for l in range(1, L):
                dsum = dsum + ms[l]
            dsum = dsum + jnp.full((16,), 1e-8, jnp.float32)
            invv = jnp.full((16,), 1.0, jnp.float32) / dsum
            for b2 in range(16):
                b = g16 * 16 + b2
                r0 = b * L
                accs = [jnp.zeros((16,), jnp.float32) for _ in range(4)]
                for l in range(L):
                    w = ms[l][b2]
                    for g in range(4):
                        accs[g] = accs[g] + w * rows_v[buf, r0 + l,
                                                       pl.ds(g * 16, 16)]
                inv = invv[b2]
                for g in range(4):
                    tm_v[buf, b, pl.ds(g * 16, 16)] = accs[g] * inv
            return bc

        lax.fori_loop(0, CB // 16, gbody, 0)
        pltpu.sync_copy(lrows_v.at[buf],
                        ids_out.at[pl.ds(base, CB), pl.ds(0, D)])
        pltpu.sync_copy(tm_v.at[buf],
                        ids_out.at[pl.ds(base, CB), pl.ds(D, D)])

    fire(0, 0)

    def pair(g, carry):
        c0 = 2 * g
        c1 = 2 * g + 1
        fire(1, c1)
        drain(0, c0)
        compute(0, c0)

        @pl.when(c1 + 1 < NCH)
        def _():
            fire(0, c1 + 1)

        drain(1, c1)
        compute(1, c1)
        return carry

    lax.fori_loop(0, NCH // 2, pair, 0)


_sc_pool = pl.kernel(
    _sc_body,
    out_type=jax.ShapeDtypeStruct((B, 2 * D), jnp.float32),
    mesh=plsc.VectorSubcoreMesh(core_axis_name="c", subcore_axis_name="s"),
    compiler_params=pltpu.CompilerParams(use_tc_tiling_on_sc=False),
    scratch_types=[
        pltpu.VMEM((PB * L,), jnp.int32),
        pltpu.VMEM((L, PB), jnp.float32),
        pltpu.VMEM((PB,), jnp.int32),
        pltpu.VMEM((2, RPC, D), jnp.float32),
        pltpu.VMEM((2, CB, D), jnp.float32),
        pltpu.VMEM((2, CB, D), jnp.float32),
        pltpu.SemaphoreType.DMA((2,)),
    ],
)


def _tc_body(a_ref, ids_ref, wc_ref, wi_ref, o_ref):
    dn = (((1,), (1,)), ((), ()))
    e = lax.dot_general(wc_ref[...], a_ref[...], dn,
                        preferred_element_type=jnp.float32)
    e = e + lax.dot_general(wi_ref[...], ids_ref[...], dn,
                            preferred_element_type=jnp.float32)
    n = jnp.sqrt(jnp.sum(e * e, axis=0, keepdims=True))
    o_ref[...] = e / jnp.maximum(n, 1e-12)


_BLK = 4096


def _tc_dense(A_content, ids, wc, wi):
    grid = (B // _BLK,)
    return pl.pallas_call(
        _tc_body,
        grid=grid,
        in_specs=[
            pl.BlockSpec((_BLK, DC), lambda i: (i, 0)),
            pl.BlockSpec((_BLK, 2 * D), lambda i: (i, 0)),
            pl.BlockSpec((TOK, DC), lambda i: (0, 0)),
            pl.BlockSpec((TOK, 2 * D), lambda i: (0, 0)),
        ],
        out_specs=pl.BlockSpec((TOK, _BLK), lambda i: (0, i)),
        out_shape=jax.ShapeDtypeStruct((TOK, B), jnp.float32),
    )(A_content, ids, wc, wi)


def kernel(A_content, tool_idx_pad, tool_mask, llm_idx, emb_tool, emb_llm,
           W_content, W_ids):
    idxf = tool_idx_pad.astype(jnp.int32).reshape(B * L)
    maskT = tool_mask.T
    llmi = llm_idx.astype(jnp.int32)
    ids = _sc_pool(idxf, maskT, llmi, emb_tool, emb_llm)
    eT = _tc_dense(A_content, ids, W_content, W_ids)
    return eT.T


# speedup vs baseline: 1.3732x; 1.3732x over previous
import jax
import jax.numpy as jnp
from jax import lax
from jax.experimental import pallas as pl
from jax.experimental.pallas import tpu as pltpu
from jax.experimental.pallas import tpu_sc as plsc

B = 16384
L = 20
D = 64
DC = 128
TOK = 64

NC = 2
NS = 16
NW = NC * NS
PB = B // NW
HB = PB // 2
CB = 32
NCH = PB // CB
RPC = CB * L
GID = 128
NG = RPC // GID


def _sc_body(idx_hbm, mask_hbm, llmidx_hbm, tool_tab, llm_tab,
             ids_out,
             stgi_v, stgm_v, idxb_v, maskb_v, lidx_v, rows_v, tm_v, lrows_v,
             gsems):
    c = lax.axis_index("c")
    s = lax.axis_index("s")
    wid = s * NC + c
    wbase = wid * PB

    pltpu.sync_copy(llmidx_hbm.at[pl.ds(wbase, PB)], lidx_v)

    iota_l = lax.iota(jnp.int32, 16) * L
    for half in range(2):
        hb = half * HB
        pltpu.sync_copy(idx_hbm.at[:, pl.ds(wbase + hb, HB)], stgi_v)
        pltpu.sync_copy(mask_hbm.at[:, pl.ds(wbase + hb, HB)], stgm_v)

        def tbody(k, tc):
            bvec = iota_l + (hb + k * 16) * L
            for l in range(L):
                plsc.store_scatter(idxb_v, (bvec + l,),
                                   stgi_v[l, pl.ds(k * 16, 16)])
                plsc.store_scatter(maskb_v, (bvec + l,),
                                   stgm_v[l, pl.ds(k * 16, 16)])
            return tc

        lax.fori_loop(0, HB // 16, tbody, 0)

    def fire(buf, ci):
        o = ci * RPC
        for j in range(NG):
            pltpu.async_copy(
                tool_tab.at[idxb_v.at[pl.ds(o + j * GID, GID)]],
                rows_v.at[buf, pl.ds(j * GID, GID)], gsems.at[buf])
        pltpu.async_copy(llm_tab.at[lidx_v.at[pl.ds(ci * CB, CB)]],
                         lrows_v.at[buf], gsems.at[buf])

    def drain(buf, ci):
        o = ci * RPC
        for j in range(NG):
            pltpu.make_async_copy(
                tool_tab.at[idxb_v.at[pl.ds(o + j * GID, GID)]],
                rows_v.at[buf, pl.ds(j * GID, GID)], gsems.at[buf]).wait()
        pltpu.make_async_copy(llm_tab.at[lidx_v.at[pl.ds(ci * CB, CB)]],
                              lrows_v.at[buf], gsems.at[buf]).wait()

    def compute(buf, ci):
        base = wbase + ci * CB
        mo = ci * RPC

        def bbody(i, bc):
            for b2 in range(2):
                b = i * 2 + b2
                r0 = b * L
                m0 = maskb_v[pl.ds(mo + r0, 16)]
                m1 = maskb_v[pl.ds(mo + r0 + 16, 16)]
                ws = ([m0[l] for l in range(16)]
                      + [m1[l] for l in range(L - 16)])
                denom = jnp.float32(1e-8)
                for l in range(L):
                    denom = denom + ws[l]
                inv = jnp.full((16,), 1.0, jnp.float32) / jnp.broadcast_to(
                    denom, (16,))
                accs = [jnp.zeros((16,), jnp.float32) for _ in range(4)]
                for l in range(L):
                    for g in range(4):
                        accs[g] = accs[g] + ws[l] * rows_v[buf, r0 + l,
                                                           pl.ds(g * 16, 16)]
                for g in range(4):
                    tm_v[buf, b, pl.ds(g * 16, 16)] = accs[g] * inv
            return bc

        lax.fori_loop(0, CB // 2, bbody, 0)
        pltpu.sync_copy(lrows_v.at[buf],
                        ids_out.at[pl.ds(base, CB), pl.ds(0, D)])
        pltpu.sync_copy(tm_v.at[buf],
                        ids_out.at[pl.ds(base, CB), pl.ds(D, D)])

    fire(0, 0)

    def pair(g, carry):
        c0 = 2 * g
        c1 = 2 * g + 1
        fire(1, c1)
        drain(0, c0)
        compute(0, c0)

        @pl.when(c1 + 1 < NCH)
        def _():
            fire(0, c1 + 1)

        drain(1, c1)
        compute(1, c1)
        return carry

    lax.fori_loop(0, NCH // 2, pair, 0)


_sc_pool = pl.kernel(
    _sc_body,
    out_type=jax.ShapeDtypeStruct((B, 2 * D), jnp.float32),
    mesh=plsc.VectorSubcoreMesh(core_axis_name="c", subcore_axis_name="s"),
    compiler_params=pltpu.CompilerParams(use_tc_tiling_on_sc=False,
                                         needs_layout_passes=False),
    scratch_types=[
        pltpu.VMEM((L, HB), jnp.int32),
        pltpu.VMEM((L, HB), jnp.float32),
        pltpu.VMEM((PB * L,), jnp.int32),
        pltpu.VMEM((PB * L + 16,), jnp.float32),
        pltpu.VMEM((PB,), jnp.int32),
        pltpu.VMEM((2, RPC, D), jnp.float32),
        pltpu.VMEM((2, CB, D), jnp.float32),
        pltpu.VMEM((2, CB, D), jnp.float32),
        pltpu.SemaphoreType.DMA((2,)),
    ],
)


def _tc_body(a_ref, ids_ref, wc_ref, wi_ref, o_ref):
    dn = (((1,), (1,)), ((), ()))
    e = lax.dot_general(wc_ref[...], a_ref[...], dn,
                        preferred_element_type=jnp.float32)
    e = e + lax.dot_general(wi_ref[...], ids_ref[...], dn,
                            preferred_element_type=jnp.float32)
    n = jnp.sqrt(jnp.sum(e * e, axis=0, keepdims=True))
    o_ref[...] = e / jnp.maximum(n, 1e-12)


_BLK = 4096


def _tc_dense(A_content, ids, wc, wi):
    grid = (B // _BLK,)
    return pl.pallas_call(
        _tc_body,
        grid=grid,
        in_specs=[
            pl.BlockSpec((_BLK, DC), lambda i: (i, 0)),
            pl.BlockSpec((_BLK, 2 * D), lambda i: (i, 0)),
            pl.BlockSpec((TOK, DC), lambda i: (0, 0)),
            pl.BlockSpec((TOK, 2 * D), lambda i: (0, 0)),
        ],
        out_specs=pl.BlockSpec((TOK, _BLK), lambda i: (0, i)),
        out_shape=jax.ShapeDtypeStruct((TOK, B), jnp.float32),
    )(A_content, ids, wc, wi)


def kernel(A_content, tool_idx_pad, tool_mask, llm_idx, emb_tool, emb_llm,
           W_content, W_ids):
    idxT = tool_idx_pad.astype(jnp.int32).T
    maskT = tool_mask.T
    llmi = llm_idx.astype(jnp.int32)
    ids = _sc_pool(idxT, maskT, llmi, emb_tool, emb_llm)
    eT = _tc_dense(A_content, ids, W_content, W_ids)
    return eT.T


# speedup vs baseline: 1.4123x; 1.0285x over previous
import jax
import jax.numpy as jnp
from jax import lax
from jax.experimental import pallas as pl
from jax.experimental.pallas import tpu as pltpu
from jax.experimental.pallas import tpu_sc as plsc

B = 16384
L = 20
D = 64
DC = 128
TOK = 64

NC = 2
NS = 16
NW = NC * NS
PB = B // NW
HB = PB // 2
CB = 32
NCH = PB // CB
RPC = CB * L
GID = 128
NG = RPC // GID


def _sc_body(idx_hbm, mask_hbm, llmidx_hbm, tool_tab, llm_tab,
             ids_out,
             stgi_v, stgm_v, idxb_v, maskb_v, lidx_v, rows_v, tm_v, lrows_v,
             gsems):
    c = lax.axis_index("c")
    s = lax.axis_index("s")
    wid = s * NC + c
    wbase = wid * PB

    pltpu.sync_copy(llmidx_hbm.at[pl.ds(wbase, PB)], lidx_v)

    iota_l = lax.iota(jnp.int32, 16) * L

    def mk_tbody(hb):
        def tbody(k, tc):
            bvec = iota_l + (hb + k * 16) * L
            for l in range(L):
                plsc.store_scatter(idxb_v, (bvec + l,),
                                   stgi_v[l, pl.ds(k * 16, 16)])
                plsc.store_scatter(maskb_v, (bvec + l,),
                                   stgm_v[l, pl.ds(k * 16, 16)])
            return tc

        return tbody

    def fire(buf, ci):
        o = ci * RPC
        for j in range(NG):
            pltpu.async_copy(
                tool_tab.at[idxb_v.at[pl.ds(o + j * GID, GID)]],
                rows_v.at[buf, pl.ds(j * GID, GID)], gsems.at[buf])
        pltpu.async_copy(llm_tab.at[lidx_v.at[pl.ds(ci * CB, CB)]],
                         lrows_v.at[buf], gsems.at[buf])

    def drain(buf, ci):
        o = ci * RPC
        for j in range(NG):
            pltpu.make_async_copy(
                tool_tab.at[idxb_v.at[pl.ds(o + j * GID, GID)]],
                rows_v.at[buf, pl.ds(j * GID, GID)], gsems.at[buf]).wait()
        pltpu.make_async_copy(llm_tab.at[lidx_v.at[pl.ds(ci * CB, CB)]],
                              lrows_v.at[buf], gsems.at[buf]).wait()

    def compute(buf, ci):
        base = wbase + ci * CB
        mo = ci * RPC

        def bbody(i, bc):
            for b2 in range(2):
                b = i * 2 + b2
                r0 = b * L
                m0 = maskb_v[pl.ds(mo + r0, 16)]
                m1 = maskb_v[pl.ds(mo + r0 + 16, 16)]
                ws = ([m0[l] for l in range(16)]
                      + [m1[l] for l in range(L - 16)])
                denom = jnp.float32(1e-8)
                for l in range(L):
                    denom = denom + ws[l]
                inv = jnp.full((16,), 1.0, jnp.float32) / jnp.broadcast_to(
                    denom, (16,))
                accs = [jnp.zeros((16,), jnp.float32) for _ in range(4)]
                for l in range(L):
                    for g in range(4):
                        accs[g] = accs[g] + ws[l] * rows_v[buf, r0 + l,
                                                           pl.ds(g * 16, 16)]
                for g in range(4):
                    tm_v[buf, b, pl.ds(g * 16, 16)] = accs[g] * inv
            return bc

        lax.fori_loop(0, CB // 2, bbody, 0)
        pltpu.sync_copy(lrows_v.at[buf],
                        ids_out.at[pl.ds(base, CB), pl.ds(0, D)])
        pltpu.sync_copy(tm_v.at[buf],
                        ids_out.at[pl.ds(base, CB), pl.ds(D, D)])

    pltpu.sync_copy(idx_hbm.at[:, pl.ds(wbase, HB)], stgi_v)
    pltpu.sync_copy(mask_hbm.at[:, pl.ds(wbase, HB)], stgm_v)
    lax.fori_loop(0, CB // 16, mk_tbody(0), 0)
    fire(0, 0)
    lax.fori_loop(CB // 16, HB // 16, mk_tbody(0), 0)
    pltpu.sync_copy(idx_hbm.at[:, pl.ds(wbase + HB, HB)], stgi_v)
    pltpu.sync_copy(mask_hbm.at[:, pl.ds(wbase + HB, HB)], stgm_v)
    lax.fori_loop(0, HB // 16, mk_tbody(HB), 0)

    def pair(g, carry):
        c0 = 2 * g
        c1 = 2 * g + 1
        fire(1, c1)
        drain(0, c0)
        compute(0, c0)

        @pl.when(c1 + 1 < NCH)
        def _():
            fire(0, c1 + 1)

        drain(1, c1)
        compute(1, c1)
        return carry

    lax.fori_loop(0, NCH // 2, pair, 0)


_sc_pool = pl.kernel(
    _sc_body,
    out_type=jax.ShapeDtypeStruct((B, 2 * D), jnp.float32),
    mesh=plsc.VectorSubcoreMesh(core_axis_name="c", subcore_axis_name="s"),
    compiler_params=pltpu.CompilerParams(use_tc_tiling_on_sc=False,
                                         needs_layout_passes=False),
    scratch_types=[
        pltpu.VMEM((L, HB), jnp.int32),
        pltpu.VMEM((L, HB), jnp.float32),
        pltpu.VMEM((PB * L,), jnp.int32),
        pltpu.VMEM((PB * L + 16,), jnp.float32),
        pltpu.VMEM((PB,), jnp.int32),
        pltpu.VMEM((2, RPC, D), jnp.float32),
        pltpu.VMEM((2, CB, D), jnp.float32),
        pltpu.VMEM((2, CB, D), jnp.float32),
        pltpu.SemaphoreType.DMA((2,)),
    ],
)


def _tc_body(a_ref, ids_ref, wc_ref, wi_ref, o_ref):
    dn = (((1,), (1,)), ((), ()))
    e = lax.dot_general(wc_ref[...], a_ref[...], dn,
                        preferred_element_type=jnp.float32)
    e = e + lax.dot_general(wi_ref[...], ids_ref[...], dn,
                            preferred_element_type=jnp.float32)
    n = jnp.sqrt(jnp.sum(e * e, axis=0, keepdims=True))
    o_ref[...] = e / jnp.maximum(n, 1e-12)


_BLK = 4096


def _tc_dense(A_content, ids, wc, wi):
    grid = (B // _BLK,)
    return pl.pallas_call(
        _tc_body,
        grid=grid,
        in_specs=[
            pl.BlockSpec((_BLK, DC), lambda i: (i, 0)),
            pl.BlockSpec((_BLK, 2 * D), lambda i: (i, 0)),
            pl.BlockSpec((TOK, DC), lambda i: (0, 0)),
            pl.BlockSpec((TOK, 2 * D), lambda i: (0, 0)),
        ],
        out_specs=pl.BlockSpec((TOK, _BLK), lambda i: (0, i)),
        out_shape=jax.ShapeDtypeStruct((TOK, B), jnp.float32),
    )(A_content, ids, wc, wi)


def kernel(A_content, tool_idx_pad, tool_mask, llm_idx, emb_tool, emb_llm,
           W_content, W_ids):
    idxT = tool_idx_pad.astype(jnp.int32).T
    maskT = tool_mask.T
    llmi = llm_idx.astype(jnp.int32)
    ids = _sc_pool(idxT, maskT, llmi, emb_tool, emb_llm)
    eT = _tc_dense(A_content, ids, W_content, W_ids)
    return eT.T
